# Initial kernel scaffold; baseline (speedup 1.0000x reference)
import jax
import jax.numpy as jnp
from jax.experimental import pallas as pl


def kernel(input, weights):
    raise NotImplementedError("write your pallas kernel here")


# speedup vs baseline: 1.6011x; 1.6011x over previous
import functools

import jax
import jax.numpy as jnp
from jax import lax
from jax.experimental import pallas as pl
from jax.experimental.pallas import tpu as pltpu
from jax.experimental.pallas import tpu_sc as plsc

EMBED_DIM = 1024
PAD = 1
BATCH = 4
SEQ = 8192
NROWS = BATCH * SEQ
NWORKERS = 32
RPW = NROWS // NWORKERS
CHUNK = 64
LANES = 16

_mesh = plsc.VectorSubcoreMesh(core_axis_name="c", subcore_axis_name="s")


@functools.partial(
    pl.kernel,
    mesh=_mesh,
    out_type=jax.ShapeDtypeStruct((NROWS, EMBED_DIM), jnp.float32),
    scratch_types=[
        pltpu.VMEM((RPW,), jnp.int32),
        pltpu.VMEM((RPW,), jnp.int32),
        pltpu.VMEM((CHUNK, EMBED_DIM), jnp.float32),
        pltpu.SemaphoreType.DMA,
    ],
)
def _emb_gather(inp_hbm, w_hbm, out_hbm, inp_v, idx_v, rows_v, sem):
    wid = lax.axis_index("s") * 2 + lax.axis_index("c")
    base = wid * RPW

    pltpu.sync_copy(inp_hbm.at[pl.ds(base, RPW)], inp_v)

    def compute_idx(j, carry):
        off = j * LANES
        v = inp_v[pl.ds(off, LANES)]
        s = (base + off + lax.iota(jnp.int32, LANES)) & (SEQ - 1)
        pad_vec = jnp.full((LANES,), PAD, jnp.int32)
        idx_v[pl.ds(off, LANES)] = jnp.where(v != PAD, s + (PAD + 1), pad_vec)
        return carry

    lax.fori_loop(0, RPW // LANES, compute_idx, 0)

    def do_chunk(i, carry):
        off = i * CHUNK
        pltpu.async_copy(w_hbm.at[idx_v.at[pl.ds(off, CHUNK)]], rows_v, sem).wait()
        pltpu.sync_copy(rows_v, out_hbm.at[pl.ds(base + off, CHUNK)])
        return carry

    lax.fori_loop(0, RPW // CHUNK, do_chunk, 0)


def kernel(input, weights):
    out = _emb_gather(input.reshape(-1), weights)
    return out.reshape(BATCH, SEQ, EMBED_DIM)


# speedup vs baseline: 1.6678x; 1.0417x over previous
import functools

import jax
import jax.numpy as jnp
from jax import lax
from jax.experimental import pallas as pl
from jax.experimental.pallas import tpu as pltpu
from jax.experimental.pallas import tpu_sc as plsc

EMBED_DIM = 1024
PAD = 1
BATCH = 4
SEQ = 8192
NROWS = BATCH * SEQ
NWORKERS = 32
RPW = NROWS // NWORKERS
CHUNK = 32
NCHUNK = RPW // CHUNK
NPAIR = NCHUNK // 2
LANES = 16

_mesh = plsc.VectorSubcoreMesh(core_axis_name="c", subcore_axis_name="s")


@functools.partial(
    pl.kernel,
    mesh=_mesh,
    out_type=jax.ShapeDtypeStruct((NROWS, EMBED_DIM), jnp.float32),
    scratch_types=[
        pltpu.VMEM((RPW,), jnp.int32),
        pltpu.VMEM((RPW,), jnp.int32),
        pltpu.VMEM((CHUNK, EMBED_DIM), jnp.float32),
        pltpu.VMEM((CHUNK, EMBED_DIM), jnp.float32),
        pltpu.SemaphoreType.DMA,
        pltpu.SemaphoreType.DMA,
        pltpu.SemaphoreType.DMA,
        pltpu.SemaphoreType.DMA,
    ],
)
def _emb_gather(inp_hbm, w_hbm, out_hbm, inp_v, idx_v, rows0, rows1,
                gsem0, gsem1, wsem0, wsem1):
    wid = lax.axis_index("s") * 2 + lax.axis_index("c")
    base = wid * RPW

    pltpu.sync_copy(inp_hbm.at[pl.ds(base, RPW)], inp_v)

    def compute_idx(j, carry):
        off = j * LANES
        v = inp_v[pl.ds(off, LANES)]
        s = (base + off + lax.iota(jnp.int32, LANES)) & (SEQ - 1)
        pad_vec = jnp.full((LANES,), PAD, jnp.int32)
        idx_v[pl.ds(off, LANES)] = jnp.where(v != PAD, s + (PAD + 1), pad_vec)
        return carry

    lax.fori_loop(0, RPW // LANES, compute_idx, 0)

    def gather_start(off, rows, gsem):
        return pltpu.async_copy(w_hbm.at[idx_v.at[pl.ds(off, CHUNK)]], rows, gsem)

    def gather_wait(off, rows, gsem):
        pltpu.make_async_copy(w_hbm.at[idx_v.at[pl.ds(off, CHUNK)]], rows, gsem).wait()

    def write_start(off, rows, wsem):
        return pltpu.async_copy(rows, out_hbm.at[pl.ds(base + off, CHUNK)], wsem)

    def write_wait(off, rows, wsem):
        pltpu.make_async_copy(rows, out_hbm.at[pl.ds(base + off, CHUNK)], wsem).wait()

    gather_start(0, rows0, gsem0)
    gather_start(CHUNK, rows1, gsem1)

    def do_pair(g, carry):
        off0 = (2 * g) * CHUNK
        off1 = off0 + CHUNK
        gather_wait(off0, rows0, gsem0)
        write_start(off0, rows0, wsem0)
        gather_wait(off1, rows1, gsem1)
        write_start(off1, rows1, wsem1)

        @pl.when(g < NPAIR - 1)
        def _refill():
            write_wait(off0, rows0, wsem0)
            gather_start(off0 + 2 * CHUNK, rows0, gsem0)
            write_wait(off1, rows1, wsem1)
            gather_start(off1 + 2 * CHUNK, rows1, gsem1)

        return carry

    lax.fori_loop(0, NPAIR, do_pair, 0)

    write_wait((NCHUNK - 2) * CHUNK, rows0, wsem0)
    write_wait((NCHUNK - 1) * CHUNK, rows1, wsem1)


def kernel(input, weights):
    out = _emb_gather(input.reshape(-1), weights)
    return out.reshape(BATCH, SEQ, EMBED_DIM)


# speedup vs baseline: 2.1000x; 1.2591x over previous
import functools

import jax
import jax.numpy as jnp
from jax import lax
from jax.experimental import pallas as pl
from jax.experimental.pallas import tpu as pltpu
from jax.experimental.pallas import tpu_sc as plsc

EMBED_DIM = 1024
PAD = 1
BATCH = 4
SEQ = 8192
NROWS = BATCH * SEQ
NWORKERS = 32
SPW = SEQ // NWORKERS
CHUNK = 32
NCHUNK = SPW // CHUNK
NPAIR = NCHUNK // 2
LANES = 16
KFIX = 16

_mesh = plsc.VectorSubcoreMesh(core_axis_name="c", subcore_axis_name="s")


@functools.partial(
    pl.kernel,
    mesh=_mesh,
    out_type=jax.ShapeDtypeStruct((NROWS, EMBED_DIM), jnp.float32),
    scratch_types=[
        pltpu.VMEM((SPW,), jnp.int32),
        pltpu.VMEM((KFIX,), jnp.int32),
        pltpu.VMEM((KFIX,), jnp.int32),
        pltpu.VMEM((CHUNK, EMBED_DIM), jnp.float32),
        pltpu.VMEM((CHUNK, EMBED_DIM), jnp.float32),
        pltpu.VMEM((KFIX, EMBED_DIM), jnp.float32),
        pltpu.SemaphoreType.DMA,
        pltpu.SemaphoreType.DMA,
        pltpu.SemaphoreType.DMA,
        pltpu.SemaphoreType.DMA,
        pltpu.SemaphoreType.DMA,
    ],
)
def _emb_stream(w_hbm, fixdst_hbm, fixsrc_hbm, out_hbm, idx_v, fdst_v, fsrc_v,
                rows0, rows1, fixbuf, gsem0, gsem1, wsem0, wsem1, fsem):
    wid = lax.axis_index("s") * 2 + lax.axis_index("c")
    seq0 = wid * SPW

    pltpu.async_copy(fixdst_hbm.at[pl.ds(wid * KFIX, KFIX)], fdst_v, fsem)
    pltpu.async_copy(fixsrc_hbm.at[pl.ds(wid * KFIX, KFIX)], fsrc_v, fsem)

    def compute_idx(j, carry):
        off = j * LANES
        idx_v[pl.ds(off, LANES)] = (
            seq0 + off + (PAD + 1) + lax.iota(jnp.int32, LANES))
        return carry

    lax.fori_loop(0, SPW // LANES, compute_idx, 0)

    def gather_start(c, rows, gsem):
        return pltpu.async_copy(w_hbm.at[idx_v.at[pl.ds(c * CHUNK, CHUNK)]],
                                rows, gsem)

    def gather_wait(c, rows, gsem):
        pltpu.make_async_copy(w_hbm.at[idx_v.at[pl.ds(c * CHUNK, CHUNK)]],
                              rows, gsem).wait()

    def write_start(c, rows, wsem):
        for b in range(BATCH):
            pltpu.async_copy(
                rows, out_hbm.at[pl.ds(b * SEQ + seq0 + c * CHUNK, CHUNK)], wsem)

    def write_wait(c, rows, wsem):
        for b in range(BATCH):
            pltpu.make_async_copy(
                rows, out_hbm.at[pl.ds(b * SEQ + seq0 + c * CHUNK, CHUNK)],
                wsem).wait()

    gather_start(0, rows0, gsem0)
    gather_start(1, rows1, gsem1)

    def do_pair(g, carry):
        c0 = 2 * g
        c1 = c0 + 1
        gather_wait(c0, rows0, gsem0)
        write_start(c0, rows0, wsem0)
        gather_wait(c1, rows1, gsem1)
        write_start(c1, rows1, wsem1)

        @pl.when(g < NPAIR - 1)
        def _refill():
            write_wait(c0, rows0, wsem0)
            gather_start(c0 + 2, rows0, gsem0)
            write_wait(c1, rows1, wsem1)
            gather_start(c1 + 2, rows1, gsem1)

        return carry

    lax.fori_loop(0, NPAIR, do_pair, 0)

    write_wait(NCHUNK - 2, rows0, wsem0)
    write_wait(NCHUNK - 1, rows1, wsem1)

    pltpu.make_async_copy(fixdst_hbm.at[pl.ds(wid * KFIX, KFIX)],
                          fdst_v, fsem).wait()
    pltpu.make_async_copy(fixsrc_hbm.at[pl.ds(wid * KFIX, KFIX)],
                          fsrc_v, fsem).wait()
    pltpu.async_copy(w_hbm.at[fsrc_v], fixbuf, fsem).wait()
    pltpu.async_copy(fixbuf, out_hbm.at[fdst_v], fsem).wait()


RPW = NROWS // NWORKERS
FCHUNK = 32
FNCHUNK = RPW // FCHUNK
FNPAIR = FNCHUNK // 2


@functools.partial(
    pl.kernel,
    mesh=_mesh,
    out_type=jax.ShapeDtypeStruct((NROWS, EMBED_DIM), jnp.float32),
    scratch_types=[
        pltpu.VMEM((RPW,), jnp.int32),
        pltpu.VMEM((RPW,), jnp.int32),
        pltpu.VMEM((FCHUNK, EMBED_DIM), jnp.float32),
        pltpu.VMEM((FCHUNK, EMBED_DIM), jnp.float32),
        pltpu.SemaphoreType.DMA,
        pltpu.SemaphoreType.DMA,
        pltpu.SemaphoreType.DMA,
        pltpu.SemaphoreType.DMA,
    ],
)
def _emb_gather_full(inp_hbm, w_hbm, out_hbm, inp_v, idx_v, rows0, rows1,
                     gsem0, gsem1, wsem0, wsem1):
    wid = lax.axis_index("s") * 2 + lax.axis_index("c")
    base = wid * RPW

    pltpu.sync_copy(inp_hbm.at[pl.ds(base, RPW)], inp_v)

    def compute_idx(j, carry):
        off = j * LANES
        v = inp_v[pl.ds(off, LANES)]
        s = (base + off + lax.iota(jnp.int32, LANES)) & (SEQ - 1)
        pad_vec = jnp.full((LANES,), PAD, jnp.int32)
        idx_v[pl.ds(off, LANES)] = jnp.where(v != PAD, s + (PAD + 1), pad_vec)
        return carry

    lax.fori_loop(0, RPW // LANES, compute_idx, 0)

    def gather_start(c, rows, gsem):
        return pltpu.async_copy(w_hbm.at[idx_v.at[pl.ds(c * FCHUNK, FCHUNK)]],
                                rows, gsem)

    def gather_wait(c, rows, gsem):
        pltpu.make_async_copy(w_hbm.at[idx_v.at[pl.ds(c * FCHUNK, FCHUNK)]],
                              rows, gsem).wait()

    def write_start(c, rows, wsem):
        pltpu.async_copy(rows, out_hbm.at[pl.ds(base + c * FCHUNK, FCHUNK)],
                         wsem)

    def write_wait(c, rows, wsem):
        pltpu.make_async_copy(rows, out_hbm.at[pl.ds(base + c * FCHUNK, FCHUNK)],
                              wsem).wait()

    gather_start(0, rows0, gsem0)
    gather_start(1, rows1, gsem1)

    def do_pair(g, carry):
        c0 = 2 * g
        c1 = c0 + 1
        gather_wait(c0, rows0, gsem0)
        write_start(c0, rows0, wsem0)
        gather_wait(c1, rows1, gsem1)
        write_start(c1, rows1, wsem1)

        @pl.when(g < FNPAIR - 1)
        def _refill():
            write_wait(c0, rows0, wsem0)
            gather_start(c0 + 2, rows0, gsem0)
            write_wait(c1, rows1, wsem1)
            gather_start(c1 + 2, rows1, gsem1)

        return carry

    lax.fori_loop(0, FNPAIR, do_pair, 0)

    write_wait(FNCHUNK - 2, rows0, wsem0)
    write_wait(FNCHUNK - 1, rows1, wsem1)


def kernel(input, weights):
    inp_flat = input.reshape(-1)

    padded = (input != PAD).astype(jnp.int32)
    blocks = (1 - padded).reshape(BATCH, NWORKERS, SPW).transpose(1, 0, 2)
    blocks = blocks.reshape(NWORKERS, BATCH * SPW)
    counts = jnp.sum(blocks, axis=1)
    overflow = jnp.any(counts > KFIX)
    _, topi = lax.top_k(blocks, KFIX)
    b_loc = topi // SPW
    s_loc = topi - b_loc * SPW
    seq = jnp.arange(NWORKERS, dtype=jnp.int32)[:, None] * SPW + s_loc
    fixdst = (b_loc * SEQ + seq).astype(jnp.int32).reshape(-1)
    is_pad = jnp.take_along_axis(blocks, topi, axis=1) > 0
    fixsrc = jnp.where(is_pad, PAD, seq + (PAD + 1)).astype(jnp.int32).reshape(-1)

    out = lax.cond(
        overflow,
        lambda i, w, d, s: _emb_gather_full(i, w),
        lambda i, w, d, s: _emb_stream(w, d, s),
        inp_flat, weights, fixdst, fixsrc)
    return out.reshape(BATCH, SEQ, EMBED_DIM)


# speedup vs baseline: 2.2573x; 1.0749x over previous
import functools

import jax
import jax.numpy as jnp
from jax import lax
from jax.experimental import pallas as pl
from jax.experimental.pallas import tpu as pltpu
from jax.experimental.pallas import tpu_sc as plsc

EMBED_DIM = 1024
PAD = 1
BATCH = 4
SEQ = 8192
NROWS = BATCH * SEQ
NWORKERS = 32
SPW = SEQ // NWORKERS
CHUNK = 32
NCHUNK = SPW // CHUNK
NPAIR = NCHUNK // 2
LANES = 16
KFIX = 16

_mesh = plsc.VectorSubcoreMesh(core_axis_name="c", subcore_axis_name="s")


@functools.partial(
    pl.kernel,
    mesh=_mesh,
    out_type=jax.ShapeDtypeStruct((NROWS, EMBED_DIM), jnp.float32),
    scratch_types=[
        pltpu.VMEM((SPW,), jnp.int32),
        pltpu.VMEM((KFIX,), jnp.int32),
        pltpu.VMEM((KFIX,), jnp.int32),
        pltpu.VMEM((CHUNK, EMBED_DIM), jnp.float32),
        pltpu.VMEM((CHUNK, EMBED_DIM), jnp.float32),
        pltpu.VMEM((KFIX, EMBED_DIM), jnp.float32),
        pltpu.SemaphoreType.DMA,
        pltpu.SemaphoreType.DMA,
        pltpu.SemaphoreType.DMA,
        pltpu.SemaphoreType.DMA,
        pltpu.SemaphoreType.DMA,
    ],
)
def _emb_stream(w_hbm, fixdst_hbm, fixsrc_hbm, out_hbm, idx_v, fdst_v, fsrc_v,
                rows0, rows1, fixbuf, gsem0, gsem1, wsem0, wsem1, fsem):
    wid = lax.axis_index("s") * 2 + lax.axis_index("c")
    seq0 = wid * SPW

    pltpu.async_copy(fixdst_hbm.at[pl.ds(wid * KFIX, KFIX)], fdst_v, fsem)
    pltpu.async_copy(fixsrc_hbm.at[pl.ds(wid * KFIX, KFIX)], fsrc_v, fsem)

    def compute_idx(j, carry):
        off = j * LANES
        idx_v[pl.ds(off, LANES)] = (
            seq0 + off + (PAD + 1) + lax.iota(jnp.int32, LANES))
        return carry

    lax.fori_loop(0, SPW // LANES, compute_idx, 0)

    def gather_start(c, rows, gsem):
        return pltpu.async_copy(w_hbm.at[idx_v.at[pl.ds(c * CHUNK, CHUNK)]],
                                rows, gsem)

    def gather_wait(c, rows, gsem):
        pltpu.make_async_copy(w_hbm.at[idx_v.at[pl.ds(c * CHUNK, CHUNK)]],
                              rows, gsem).wait()

    def write_start(c, rows, wsem):
        for b in range(BATCH):
            pltpu.async_copy(
                rows, out_hbm.at[pl.ds(b * SEQ + seq0 + c * CHUNK, CHUNK)], wsem)

    def write_wait(c, rows, wsem):
        for b in range(BATCH):
            pltpu.make_async_copy(
                rows, out_hbm.at[pl.ds(b * SEQ + seq0 + c * CHUNK, CHUNK)],
                wsem).wait()

    gather_start(0, rows0, gsem0)
    gather_start(1, rows1, gsem1)

    def do_pair(g, carry):
        c0 = 2 * g
        c1 = c0 + 1
        gather_wait(c0, rows0, gsem0)
        write_start(c0, rows0, wsem0)
        gather_wait(c1, rows1, gsem1)
        write_start(c1, rows1, wsem1)

        @pl.when(g < NPAIR - 1)
        def _refill():
            write_wait(c0, rows0, wsem0)
            gather_start(c0 + 2, rows0, gsem0)
            write_wait(c1, rows1, wsem1)
            gather_start(c1 + 2, rows1, gsem1)

        return carry

    lax.fori_loop(0, NPAIR, do_pair, 0)

    write_wait(NCHUNK - 2, rows0, wsem0)
    write_wait(NCHUNK - 1, rows1, wsem1)

    pltpu.make_async_copy(fixdst_hbm.at[pl.ds(wid * KFIX, KFIX)],
                          fdst_v, fsem).wait()
    pltpu.make_async_copy(fixsrc_hbm.at[pl.ds(wid * KFIX, KFIX)],
                          fsrc_v, fsem).wait()
    pltpu.async_copy(w_hbm.at[fsrc_v], fixbuf, fsem).wait()
    pltpu.async_copy(fixbuf, out_hbm.at[fdst_v], fsem).wait()


RPW = NROWS // NWORKERS
FCHUNK = 32
FNCHUNK = RPW // FCHUNK
FNPAIR = FNCHUNK // 2


@functools.partial(
    pl.kernel,
    mesh=_mesh,
    out_type=jax.ShapeDtypeStruct((NROWS, EMBED_DIM), jnp.float32),
    scratch_types=[
        pltpu.VMEM((RPW,), jnp.int32),
        pltpu.VMEM((RPW,), jnp.int32),
        pltpu.VMEM((FCHUNK, EMBED_DIM), jnp.float32),
        pltpu.VMEM((FCHUNK, EMBED_DIM), jnp.float32),
        pltpu.SemaphoreType.DMA,
        pltpu.SemaphoreType.DMA,
        pltpu.SemaphoreType.DMA,
        pltpu.SemaphoreType.DMA,
    ],
)
def _emb_gather_full(inp_hbm, w_hbm, out_hbm, inp_v, idx_v, rows0, rows1,
                     gsem0, gsem1, wsem0, wsem1):
    wid = lax.axis_index("s") * 2 + lax.axis_index("c")
    base = wid * RPW

    pltpu.sync_copy(inp_hbm.at[pl.ds(base, RPW)], inp_v)

    def compute_idx(j, carry):
        off = j * LANES
        v = inp_v[pl.ds(off, LANES)]
        s = (base + off + lax.iota(jnp.int32, LANES)) & (SEQ - 1)
        pad_vec = jnp.full((LANES,), PAD, jnp.int32)
        idx_v[pl.ds(off, LANES)] = jnp.where(v != PAD, s + (PAD + 1), pad_vec)
        return carry

    lax.fori_loop(0, RPW // LANES, compute_idx, 0)

    def gather_start(c, rows, gsem):
        return pltpu.async_copy(w_hbm.at[idx_v.at[pl.ds(c * FCHUNK, FCHUNK)]],
                                rows, gsem)

    def gather_wait(c, rows, gsem):
        pltpu.make_async_copy(w_hbm.at[idx_v.at[pl.ds(c * FCHUNK, FCHUNK)]],
                              rows, gsem).wait()

    def write_start(c, rows, wsem):
        pltpu.async_copy(rows, out_hbm.at[pl.ds(base + c * FCHUNK, FCHUNK)],
                         wsem)

    def write_wait(c, rows, wsem):
        pltpu.make_async_copy(rows, out_hbm.at[pl.ds(base + c * FCHUNK, FCHUNK)],
                              wsem).wait()

    gather_start(0, rows0, gsem0)
    gather_start(1, rows1, gsem1)

    def do_pair(g, carry):
        c0 = 2 * g
        c1 = c0 + 1
        gather_wait(c0, rows0, gsem0)
        write_start(c0, rows0, wsem0)
        gather_wait(c1, rows1, gsem1)
        write_start(c1, rows1, wsem1)

        @pl.when(g < FNPAIR - 1)
        def _refill():
            write_wait(c0, rows0, wsem0)
            gather_start(c0 + 2, rows0, gsem0)
            write_wait(c1, rows1, wsem1)
            gather_start(c1 + 2, rows1, gsem1)

        return carry

    lax.fori_loop(0, FNPAIR, do_pair, 0)

    write_wait(FNCHUNK - 2, rows0, wsem0)
    write_wait(FNCHUNK - 1, rows1, wsem1)


def kernel(input, weights):
    inp_flat = input.reshape(-1)

    padded = (input != PAD).astype(jnp.int32)
    blocks = (1 - padded).reshape(BATCH, NWORKERS, SPW).transpose(1, 0, 2)
    blocks = blocks.reshape(NWORKERS, BATCH * SPW)
    counts = jnp.sum(blocks, axis=1)
    overflow = jnp.any(counts > KFIX)
    vals, topi = lax.top_k(blocks, KFIX)
    b_loc = topi // SPW
    s_loc = topi - b_loc * SPW
    seq = jnp.arange(NWORKERS, dtype=jnp.int32)[:, None] * SPW + s_loc
    fixdst = (b_loc * SEQ + seq).astype(jnp.int32).reshape(-1)
    fixsrc = jnp.where(vals > 0, PAD, seq + (PAD + 1)).astype(jnp.int32).reshape(-1)

    out = lax.cond(
        overflow,
        lambda i, w, d, s: _emb_gather_full(i, w),
        lambda i, w, d, s: _emb_stream(w, d, s),
        inp_flat, weights, fixdst, fixsrc)
    return out.reshape(BATCH, SEQ, EMBED_DIM)


# speedup vs baseline: 2.3746x; 1.0520x over previous
import functools

import jax
import jax.numpy as jnp
from jax import lax
from jax.experimental import pallas as pl
from jax.experimental.pallas import tpu as pltpu
from jax.experimental.pallas import tpu_sc as plsc

EMBED_DIM = 1024
PAD = 1
BATCH = 4
SEQ = 8192
NROWS = BATCH * SEQ
NWORKERS = 32
SPW = SEQ // NWORKERS
CHUNK = 32
NCHUNK = SPW // CHUNK
NPAIR = NCHUNK // 2
LANES = 16
KFIX = 16

_mesh = plsc.VectorSubcoreMesh(core_axis_name="c", subcore_axis_name="s")


@functools.partial(
    pl.kernel,
    mesh=_mesh,
    out_type=jax.ShapeDtypeStruct((NROWS, EMBED_DIM), jnp.float32),
    scratch_types=[
        pltpu.VMEM((SPW,), jnp.int32),
        pltpu.VMEM((CHUNK, EMBED_DIM), jnp.float32),
        pltpu.VMEM((CHUNK, EMBED_DIM), jnp.float32),
        pltpu.SemaphoreType.DMA,
        pltpu.SemaphoreType.DMA,
        pltpu.SemaphoreType.DMA,
        pltpu.SemaphoreType.DMA,
    ],
)
def _emb_stream(w_hbm, out_hbm, idx_v, rows0, rows1, gsem0, gsem1,
                wsem0, wsem1):
    wid = lax.axis_index("s") * 2 + lax.axis_index("c")
    seq0 = wid * SPW

    def compute_idx(j, carry):
        off = j * LANES
        idx_v[pl.ds(off, LANES)] = (
            seq0 + off + (PAD + 1) + lax.iota(jnp.int32, LANES))
        return carry

    lax.fori_loop(0, SPW // LANES, compute_idx, 0)

    def gather_start(c, rows, gsem):
        return pltpu.async_copy(w_hbm.at[idx_v.at[pl.ds(c * CHUNK, CHUNK)]],
                                rows, gsem)

    def gather_wait(c, rows, gsem):
        pltpu.make_async_copy(w_hbm.at[idx_v.at[pl.ds(c * CHUNK, CHUNK)]],
                              rows, gsem).wait()

    def write_start(c, rows, wsem):
        for b in range(BATCH):
            pltpu.async_copy(
                rows, out_hbm.at[pl.ds(b * SEQ + seq0 + c * CHUNK, CHUNK)], wsem)

    def write_wait(c, rows, wsem):
        for b in range(BATCH):
            pltpu.make_async_copy(
                rows, out_hbm.at[pl.ds(b * SEQ + seq0 + c * CHUNK, CHUNK)],
                wsem).wait()

    gather_start(0, rows0, gsem0)
    gather_start(1, rows1, gsem1)

    def do_pair(g, carry):
        c0 = 2 * g
        c1 = c0 + 1
        gather_wait(c0, rows0, gsem0)
        write_start(c0, rows0, wsem0)
        gather_wait(c1, rows1, gsem1)
        write_start(c1, rows1, wsem1)

        @pl.when(g < NPAIR - 1)
        def _refill():
            write_wait(c0, rows0, wsem0)
            gather_start(c0 + 2, rows0, gsem0)
            write_wait(c1, rows1, wsem1)
            gather_start(c1 + 2, rows1, gsem1)

        return carry

    lax.fori_loop(0, NPAIR, do_pair, 0)

    write_wait(NCHUNK - 2, rows0, wsem0)
    write_wait(NCHUNK - 1, rows1, wsem1)


@functools.partial(
    pl.kernel,
    mesh=_mesh,
    out_type=(),
    scratch_types=[
        pltpu.VMEM((KFIX,), jnp.int32),
        pltpu.VMEM((KFIX,), jnp.int32),
        pltpu.VMEM((KFIX, EMBED_DIM), jnp.float32),
        pltpu.SemaphoreType.DMA,
    ],
)
def _emb_fixup(w_hbm, fixdst_hbm, fixsrc_hbm, out_hbm, fdst_v, fsrc_v,
               fixbuf, fsem):
    wid = lax.axis_index("s") * 2 + lax.axis_index("c")
    pltpu.async_copy(fixdst_hbm.at[pl.ds(wid * KFIX, KFIX)], fdst_v, fsem)
    pltpu.async_copy(fixsrc_hbm.at[pl.ds(wid * KFIX, KFIX)], fsrc_v, fsem)
    pltpu.make_async_copy(fixdst_hbm.at[pl.ds(wid * KFIX, KFIX)],
                          fdst_v, fsem).wait()
    pltpu.make_async_copy(fixsrc_hbm.at[pl.ds(wid * KFIX, KFIX)],
                          fsrc_v, fsem).wait()
    pltpu.async_copy(w_hbm.at[fsrc_v], fixbuf, fsem).wait()
    pltpu.async_copy(fixbuf, out_hbm.at[fdst_v], fsem).wait()


RPW = NROWS // NWORKERS
FCHUNK = 32
FNCHUNK = RPW // FCHUNK
FNPAIR = FNCHUNK // 2


@functools.partial(
    pl.kernel,
    mesh=_mesh,
    out_type=jax.ShapeDtypeStruct((NROWS, EMBED_DIM), jnp.float32),
    scratch_types=[
        pltpu.VMEM((RPW,), jnp.int32),
        pltpu.VMEM((RPW,), jnp.int32),
        pltpu.VMEM((FCHUNK, EMBED_DIM), jnp.float32),
        pltpu.VMEM((FCHUNK, EMBED_DIM), jnp.float32),
        pltpu.SemaphoreType.DMA,
        pltpu.SemaphoreType.DMA,
        pltpu.SemaphoreType.DMA,
        pltpu.SemaphoreType.DMA,
    ],
)
def _emb_gather_full(inp_hbm, w_hbm, out_hbm, inp_v, idx_v, rows0, rows1,
                     gsem0, gsem1, wsem0, wsem1):
    wid = lax.axis_index("s") * 2 + lax.axis_index("c")
    base = wid * RPW

    pltpu.sync_copy(inp_hbm.at[pl.ds(base, RPW)], inp_v)

    def compute_idx(j, carry):
        off = j * LANES
        v = inp_v[pl.ds(off, LANES)]
        s = (base + off + lax.iota(jnp.int32, LANES)) & (SEQ - 1)
        pad_vec = jnp.full((LANES,), PAD, jnp.int32)
        idx_v[pl.ds(off, LANES)] = jnp.where(v != PAD, s + (PAD + 1), pad_vec)
        return carry

    lax.fori_loop(0, RPW // LANES, compute_idx, 0)

    def gather_start(c, rows, gsem):
        return pltpu.async_copy(w_hbm.at[idx_v.at[pl.ds(c * FCHUNK, FCHUNK)]],
                                rows, gsem)

    def gather_wait(c, rows, gsem):
        pltpu.make_async_copy(w_hbm.at[idx_v.at[pl.ds(c * FCHUNK, FCHUNK)]],
                              rows, gsem).wait()

    def write_start(c, rows, wsem):
        pltpu.async_copy(rows, out_hbm.at[pl.ds(base + c * FCHUNK, FCHUNK)],
                         wsem)

    def write_wait(c, rows, wsem):
        pltpu.make_async_copy(rows, out_hbm.at[pl.ds(base + c * FCHUNK, FCHUNK)],
                              wsem).wait()

    gather_start(0, rows0, gsem0)
    gather_start(1, rows1, gsem1)

    def do_pair(g, carry):
        c0 = 2 * g
        c1 = c0 + 1
        gather_wait(c0, rows0, gsem0)
        write_start(c0, rows0, wsem0)
        gather_wait(c1, rows1, gsem1)
        write_start(c1, rows1, wsem1)

        @pl.when(g < FNPAIR - 1)
        def _refill():
            write_wait(c0, rows0, wsem0)
            gather_start(c0 + 2, rows0, gsem0)
            write_wait(c1, rows1, wsem1)
            gather_start(c1 + 2, rows1, gsem1)

        return carry

    lax.fori_loop(0, FNPAIR, do_pair, 0)

    write_wait(FNCHUNK - 2, rows0, wsem0)
    write_wait(FNCHUNK - 1, rows1, wsem1)


def kernel(input, weights):
    inp_flat = input.reshape(-1)

    out1 = _emb_stream(weights)

    padmask = (input == PAD).astype(jnp.int32)
    blocks = padmask.reshape(BATCH, NWORKERS, SPW).transpose(1, 0, 2)
    blocks = blocks.reshape(NWORKERS, BATCH * SPW)
    counts = jnp.sum(blocks, axis=1)
    overflow = jnp.any(counts > KFIX)
    vals, topi = lax.top_k(blocks, KFIX)
    b_loc = topi // SPW
    s_loc = topi - b_loc * SPW
    seq = jnp.arange(NWORKERS, dtype=jnp.int32)[:, None] * SPW + s_loc
    fixdst = (b_loc * SEQ + seq).astype(jnp.int32).reshape(-1)
    fixsrc = jnp.where(vals > 0, PAD, seq + (PAD + 1)).astype(jnp.int32).reshape(-1)

    def fast_path(operands):
        o1, w, fd, fs, _ = operands
        ref = jax.new_ref(o1)
        _emb_fixup(w, fd, fs, ref)
        return ref[...]

    def slow_path(operands):
        _, w, _, _, i = operands
        return _emb_gather_full(i, w)

    out = lax.cond(overflow, slow_path, fast_path,
                   (out1, weights, fixdst, fixsrc, inp_flat))
    return out.reshape(BATCH, SEQ, EMBED_DIM)


# speedup vs baseline: 2.4224x; 1.0201x over previous
import functools

import jax
import jax.numpy as jnp
from jax import lax
from jax.experimental import pallas as pl
from jax.experimental.pallas import tpu as pltpu
from jax.experimental.pallas import tpu_sc as plsc

EMBED_DIM = 1024
PAD = 1
BATCH = 4
SEQ = 8192
NROWS = BATCH * SEQ
NWORKERS = 32
SPW = SEQ // NWORKERS
CHUNK = 32
NCHUNK = SPW // CHUNK
NPAIR = NCHUNK // 2
LANES = 16
KFIX = 8

_mesh = plsc.VectorSubcoreMesh(core_axis_name="c", subcore_axis_name="s")


@functools.partial(
    pl.kernel,
    mesh=_mesh,
    out_type=jax.ShapeDtypeStruct((NROWS, EMBED_DIM), jnp.float32),
    scratch_types=[
        pltpu.VMEM((SPW,), jnp.int32),
        pltpu.VMEM((CHUNK, EMBED_DIM), jnp.float32),
        pltpu.VMEM((CHUNK, EMBED_DIM), jnp.float32),
        pltpu.SemaphoreType.DMA,
        pltpu.SemaphoreType.DMA,
        pltpu.SemaphoreType.DMA,
        pltpu.SemaphoreType.DMA,
    ],
)
def _emb_stream(w_hbm, out_hbm, idx_v, rows0, rows1, gsem0, gsem1,
                wsem0, wsem1):
    wid = lax.axis_index("s") * 2 + lax.axis_index("c")
    seq0 = wid * SPW

    def compute_idx(j, carry):
        off = j * LANES
        idx_v[pl.ds(off, LANES)] = (
            seq0 + off + (PAD + 1) + lax.iota(jnp.int32, LANES))
        return carry

    lax.fori_loop(0, SPW // LANES, compute_idx, 0)

    def gather_start(c, rows, gsem):
        return pltpu.async_copy(w_hbm.at[idx_v.at[pl.ds(c * CHUNK, CHUNK)]],
                                rows, gsem)

    def gather_wait(c, rows, gsem):
        pltpu.make_async_copy(w_hbm.at[idx_v.at[pl.ds(c * CHUNK, CHUNK)]],
                              rows, gsem).wait()

    def write_start(c, rows, wsem):
        def one(b, carry):
            pltpu.async_copy(
                rows, out_hbm.at[pl.ds(b * SEQ + seq0 + c * CHUNK, CHUNK)], wsem)
            return carry
        lax.fori_loop(0, BATCH, one, 0)

    def write_wait(c, rows, wsem):
        def one(b, carry):
            pltpu.make_async_copy(
                rows, out_hbm.at[pl.ds(b * SEQ + seq0 + c * CHUNK, CHUNK)],
                wsem).wait()
            return carry
        lax.fori_loop(0, BATCH, one, 0)

    gather_start(0, rows0, gsem0)
    gather_start(1, rows1, gsem1)

    def do_pair(g, carry):
        c0 = 2 * g
        c1 = c0 + 1
        gather_wait(c0, rows0, gsem0)
        write_start(c0, rows0, wsem0)
        gather_wait(c1, rows1, gsem1)
        write_start(c1, rows1, wsem1)

        @pl.when(g < NPAIR - 1)
        def _refill():
            write_wait(c0, rows0, wsem0)
            gather_start(c0 + 2, rows0, gsem0)
            write_wait(c1, rows1, wsem1)
            gather_start(c1 + 2, rows1, gsem1)

        return carry

    lax.fori_loop(0, NPAIR, do_pair, 0)

    write_wait(NCHUNK - 2, rows0, wsem0)
    write_wait(NCHUNK - 1, rows1, wsem1)


@functools.partial(
    pl.kernel,
    mesh=_mesh,
    out_type=(),
    scratch_types=[
        pltpu.VMEM((KFIX,), jnp.int32),
        pltpu.VMEM((KFIX,), jnp.int32),
        pltpu.VMEM((KFIX, EMBED_DIM), jnp.float32),
        pltpu.SemaphoreType.DMA,
    ],
)
def _emb_fixup(w_hbm, fixdst_hbm, fixsrc_hbm, out_hbm, fdst_v, fsrc_v,
               fixbuf, fsem):
    wid = lax.axis_index("s") * 2 + lax.axis_index("c")
    pltpu.async_copy(fixdst_hbm.at[pl.ds(wid * KFIX, KFIX)], fdst_v, fsem)
    pltpu.async_copy(fixsrc_hbm.at[pl.ds(wid * KFIX, KFIX)], fsrc_v, fsem)
    pltpu.make_async_copy(fixdst_hbm.at[pl.ds(wid * KFIX, KFIX)],
                          fdst_v, fsem).wait()
    pltpu.make_async_copy(fixsrc_hbm.at[pl.ds(wid * KFIX, KFIX)],
                          fsrc_v, fsem).wait()
    pltpu.async_copy(w_hbm.at[fsrc_v], fixbuf, fsem).wait()
    pltpu.async_copy(fixbuf, out_hbm.at[fdst_v], fsem).wait()


RPW = NROWS // NWORKERS
FCHUNK = 32
FNCHUNK = RPW // FCHUNK
FNPAIR = FNCHUNK // 2


@functools.partial(
    pl.kernel,
    mesh=_mesh,
    out_type=jax.ShapeDtypeStruct((NROWS, EMBED_DIM), jnp.float32),
    scratch_types=[
        pltpu.VMEM((RPW,), jnp.int32),
        pltpu.VMEM((RPW,), jnp.int32),
        pltpu.VMEM((FCHUNK, EMBED_DIM), jnp.float32),
        pltpu.VMEM((FCHUNK, EMBED_DIM), jnp.float32),
        pltpu.SemaphoreType.DMA,
        pltpu.SemaphoreType.DMA,
        pltpu.SemaphoreType.DMA,
        pltpu.SemaphoreType.DMA,
    ],
)
def _emb_gather_full(inp_hbm, w_hbm, out_hbm, inp_v, idx_v, rows0, rows1,
                     gsem0, gsem1, wsem0, wsem1):
    wid = lax.axis_index("s") * 2 + lax.axis_index("c")
    base = wid * RPW

    pltpu.sync_copy(inp_hbm.at[pl.ds(base, RPW)], inp_v)

    def compute_idx(j, carry):
        off = j * LANES
        v = inp_v[pl.ds(off, LANES)]
        s = (base + off + lax.iota(jnp.int32, LANES)) & (SEQ - 1)
        pad_vec = jnp.full((LANES,), PAD, jnp.int32)
        idx_v[pl.ds(off, LANES)] = jnp.where(v != PAD, s + (PAD + 1), pad_vec)
        return carry

    lax.fori_loop(0, RPW // LANES, compute_idx, 0)

    def gather_start(c, rows, gsem):
        return pltpu.async_copy(w_hbm.at[idx_v.at[pl.ds(c * FCHUNK, FCHUNK)]],
                                rows, gsem)

    def gather_wait(c, rows, gsem):
        pltpu.make_async_copy(w_hbm.at[idx_v.at[pl.ds(c * FCHUNK, FCHUNK)]],
                              rows, gsem).wait()

    def write_start(c, rows, wsem):
        pltpu.async_copy(rows, out_hbm.at[pl.ds(base + c * FCHUNK, FCHUNK)],
                         wsem)

    def write_wait(c, rows, wsem):
        pltpu.make_async_copy(rows, out_hbm.at[pl.ds(base + c * FCHUNK, FCHUNK)],
                              wsem).wait()

    gather_start(0, rows0, gsem0)
    gather_start(1, rows1, gsem1)

    def do_pair(g, carry):
        c0 = 2 * g
        c1 = c0 + 1
        gather_wait(c0, rows0, gsem0)
        write_start(c0, rows0, wsem0)
        gather_wait(c1, rows1, gsem1)
        write_start(c1, rows1, wsem1)

        @pl.when(g < FNPAIR - 1)
        def _refill():
            write_wait(c0, rows0, wsem0)
            gather_start(c0 + 2, rows0, gsem0)
            write_wait(c1, rows1, wsem1)
            gather_start(c1 + 2, rows1, gsem1)

        return carry

    lax.fori_loop(0, FNPAIR, do_pair, 0)

    write_wait(FNCHUNK - 2, rows0, wsem0)
    write_wait(FNCHUNK - 1, rows1, wsem1)


def kernel(input, weights):
    inp_flat = input.reshape(-1)

    out1 = _emb_stream(weights)

    padmask = (input == PAD).astype(jnp.int32)
    blocks = padmask.reshape(BATCH, NWORKERS, SPW).transpose(1, 0, 2)
    blocks = blocks.reshape(NWORKERS, BATCH * SPW)
    counts = jnp.sum(blocks, axis=1)
    overflow = jnp.any(counts > KFIX)
    vals, topi = lax.top_k(blocks, KFIX)
    b_loc = topi // SPW
    s_loc = topi - b_loc * SPW
    seq = jnp.arange(NWORKERS, dtype=jnp.int32)[:, None] * SPW + s_loc
    fixdst = (b_loc * SEQ + seq).astype(jnp.int32).reshape(-1)
    fixsrc = jnp.where(vals > 0, PAD, seq + (PAD + 1)).astype(jnp.int32).reshape(-1)

    ref = jax.new_ref(out1)
    _emb_fixup(weights, fixdst, fixsrc, ref)
    out1_fixed = ref[...]

    out = lax.cond(
        overflow,
        lambda o, w, i: _emb_gather_full(i, w),
        lambda o, w, i: o,
        out1_fixed, weights, inp_flat)
    return out.reshape(BATCH, SEQ, EMBED_DIM)


# speedup vs baseline: 2.4915x; 1.0285x over previous
import functools

import jax
import jax.numpy as jnp
from jax import lax
from jax.experimental import pallas as pl
from jax.experimental.pallas import tpu as pltpu
from jax.experimental.pallas import tpu_sc as plsc

EMBED_DIM = 1024
PAD = 1
BATCH = 4
SEQ = 8192
NROWS = BATCH * SEQ
NWORKERS = 32
SPW = SEQ // NWORKERS
CHUNK = 32
NCHUNK = SPW // CHUNK
NPAIR = NCHUNK // 2
LANES = 16
KFIX = 8

_mesh = plsc.VectorSubcoreMesh(core_axis_name="c", subcore_axis_name="s")


@functools.partial(
    pl.kernel,
    mesh=_mesh,
    out_type=jax.ShapeDtypeStruct((NROWS, EMBED_DIM), jnp.float32),
    scratch_types=[
        pltpu.VMEM((SPW,), jnp.int32),
        pltpu.VMEM((2 * CHUNK, EMBED_DIM), jnp.float32),
        pltpu.SemaphoreType.DMA((2,)),
        pltpu.SemaphoreType.DMA((2,)),
    ],
)
def _emb_stream(w_hbm, out_hbm, idx_v, rows, gsems, wsems):
    wid = lax.axis_index("s") * 2 + lax.axis_index("c")
    seq0 = wid * SPW

    def compute_idx(j, carry):
        off = j * LANES
        idx_v[pl.ds(off, LANES)] = (
            seq0 + off + (PAD + 1) + lax.iota(jnp.int32, LANES))
        return carry

    lax.fori_loop(0, SPW // LANES, compute_idx, 0)

    def buf(c):
        return rows.at[pl.ds((c & 1) * CHUNK, CHUNK)]

    def gather_start(c):
        return pltpu.async_copy(w_hbm.at[idx_v.at[pl.ds(c * CHUNK, CHUNK)]],
                                buf(c), gsems.at[c & 1])

    def gather_wait(c):
        pltpu.make_async_copy(w_hbm.at[idx_v.at[pl.ds(c * CHUNK, CHUNK)]],
                              buf(c), gsems.at[c & 1]).wait()

    def write_start(c):
        def one(b, carry):
            pltpu.async_copy(
                buf(c), out_hbm.at[pl.ds(b * SEQ + seq0 + c * CHUNK, CHUNK)],
                wsems.at[c & 1])
            return carry
        lax.fori_loop(0, BATCH, one, 0)

    def write_wait(c):
        def one(b, carry):
            pltpu.make_async_copy(
                buf(c), out_hbm.at[pl.ds(b * SEQ + seq0 + c * CHUNK, CHUNK)],
                wsems.at[c & 1]).wait()
            return carry
        lax.fori_loop(0, BATCH, one, 0)

    gather_start(0)
    gather_start(1)

    def do_chunk(c, carry):
        gather_wait(c)
        write_start(c)

        @pl.when(c < NCHUNK - 2)
        def _refill():
            write_wait(c)
            gather_start(c + 2)

        return carry

    lax.fori_loop(0, NCHUNK, do_chunk, 0)

    write_wait(NCHUNK - 2)
    write_wait(NCHUNK - 1)


@functools.partial(
    pl.kernel,
    mesh=_mesh,
    out_type=(),
    scratch_types=[
        pltpu.VMEM((KFIX,), jnp.int32),
        pltpu.VMEM((KFIX,), jnp.int32),
        pltpu.VMEM((KFIX, EMBED_DIM), jnp.float32),
        pltpu.SemaphoreType.DMA,
    ],
)
def _emb_fixup(w_hbm, fixdst_hbm, fixsrc_hbm, out_hbm, fdst_v, fsrc_v,
               fixbuf, fsem):
    wid = lax.axis_index("s") * 2 + lax.axis_index("c")
    pltpu.async_copy(fixdst_hbm.at[pl.ds(wid * KFIX, KFIX)], fdst_v, fsem)
    pltpu.async_copy(fixsrc_hbm.at[pl.ds(wid * KFIX, KFIX)], fsrc_v, fsem)
    pltpu.make_async_copy(fixdst_hbm.at[pl.ds(wid * KFIX, KFIX)],
                          fdst_v, fsem).wait()
    pltpu.make_async_copy(fixsrc_hbm.at[pl.ds(wid * KFIX, KFIX)],
                          fsrc_v, fsem).wait()
    pltpu.async_copy(w_hbm.at[fsrc_v], fixbuf, fsem).wait()
    pltpu.async_copy(fixbuf, out_hbm.at[fdst_v], fsem).wait()


RPW = NROWS // NWORKERS
FCHUNK = 32
FNCHUNK = RPW // FCHUNK
FNPAIR = FNCHUNK // 2


@functools.partial(
    pl.kernel,
    mesh=_mesh,
    out_type=jax.ShapeDtypeStruct((NROWS, EMBED_DIM), jnp.float32),
    scratch_types=[
        pltpu.VMEM((RPW,), jnp.int32),
        pltpu.VMEM((RPW,), jnp.int32),
        pltpu.VMEM((FCHUNK, EMBED_DIM), jnp.float32),
        pltpu.VMEM((FCHUNK, EMBED_DIM), jnp.float32),
        pltpu.SemaphoreType.DMA,
        pltpu.SemaphoreType.DMA,
        pltpu.SemaphoreType.DMA,
        pltpu.SemaphoreType.DMA,
    ],
)
def _emb_gather_full(inp_hbm, w_hbm, out_hbm, inp_v, idx_v, rows0, rows1,
                     gsem0, gsem1, wsem0, wsem1):
    wid = lax.axis_index("s") * 2 + lax.axis_index("c")
    base = wid * RPW

    pltpu.sync_copy(inp_hbm.at[pl.ds(base, RPW)], inp_v)

    def compute_idx(j, carry):
        off = j * LANES
        v = inp_v[pl.ds(off, LANES)]
        s = (base + off + lax.iota(jnp.int32, LANES)) & (SEQ - 1)
        pad_vec = jnp.full((LANES,), PAD, jnp.int32)
        idx_v[pl.ds(off, LANES)] = jnp.where(v != PAD, s + (PAD + 1), pad_vec)
        return carry

    lax.fori_loop(0, RPW // LANES, compute_idx, 0)

    def gather_start(c, rows, gsem):
        return pltpu.async_copy(w_hbm.at[idx_v.at[pl.ds(c * FCHUNK, FCHUNK)]],
                                rows, gsem)

    def gather_wait(c, rows, gsem):
        pltpu.make_async_copy(w_hbm.at[idx_v.at[pl.ds(c * FCHUNK, FCHUNK)]],
                              rows, gsem).wait()

    def write_start(c, rows, wsem):
        pltpu.async_copy(rows, out_hbm.at[pl.ds(base + c * FCHUNK, FCHUNK)],
                         wsem)

    def write_wait(c, rows, wsem):
        pltpu.make_async_copy(rows, out_hbm.at[pl.ds(base + c * FCHUNK, FCHUNK)],
                              wsem).wait()

    gather_start(0, rows0, gsem0)
    gather_start(1, rows1, gsem1)

    def do_pair(g, carry):
        c0 = 2 * g
        c1 = c0 + 1
        gather_wait(c0, rows0, gsem0)
        write_start(c0, rows0, wsem0)
        gather_wait(c1, rows1, gsem1)
        write_start(c1, rows1, wsem1)

        @pl.when(g < FNPAIR - 1)
        def _refill():
            write_wait(c0, rows0, wsem0)
            gather_start(c0 + 2, rows0, gsem0)
            write_wait(c1, rows1, wsem1)
            gather_start(c1 + 2, rows1, gsem1)

        return carry

    lax.fori_loop(0, FNPAIR, do_pair, 0)

    write_wait(FNCHUNK - 2, rows0, wsem0)
    write_wait(FNCHUNK - 1, rows1, wsem1)


def kernel(input, weights):
    inp_flat = input.reshape(-1)

    out1 = _emb_stream(weights)

    padmask = (input == PAD).astype(jnp.int32)
    blocks = padmask.reshape(BATCH, NWORKERS, SPW).transpose(1, 0, 2)
    blocks = blocks.reshape(NWORKERS, BATCH * SPW)
    counts = jnp.sum(blocks, axis=1)
    overflow = jnp.any(counts > KFIX)
    vals, topi = lax.top_k(blocks, KFIX)
    b_loc = topi // SPW
    s_loc = topi - b_loc * SPW
    seq = jnp.arange(NWORKERS, dtype=jnp.int32)[:, None] * SPW + s_loc
    fixdst = (b_loc * SEQ + seq).astype(jnp.int32).reshape(-1)
    fixsrc = jnp.where(vals > 0, PAD, seq + (PAD + 1)).astype(jnp.int32).reshape(-1)

    ref = jax.new_ref(out1)
    _emb_fixup(weights, fixdst, fixsrc, ref)
    out1_fixed = ref[...]

    out = lax.cond(
        overflow,
        lambda o, w, i: _emb_gather_full(i, w),
        lambda o, w, i: o,
        out1_fixed, weights, inp_flat)
    return out.reshape(BATCH, SEQ, EMBED_DIM)
